# baseline (speedup 1.0000x reference)
import jax
import jax.numpy as jnp
from jax.experimental import pallas as pl

N = 10000; E = 160000; N_SUB = 20000; E_SUB = 320000; B = 4; P = 32
NHID = 128; NLAYER_GNN = 2; NLAYER_MIX = 2


def _layernorm(x, g, b):
    m = x.mean(-1, keepdims=True)
    v = ((x - m) ** 2).mean(-1, keepdims=True)
    return (x - m) / jnp.sqrt(v + 1e-5) * g + b


def _ident_kernel(x_ref, o_ref):
    o_ref[...] = x_ref[...]


def kernel(x, edge_attr, combined_subgraphs, subgraphs_nodes_mapper, subgraphs_edges_mapper, subgraphs_batch, mask, W_in, b_in, W_edge, b_edge, W_g, b_g, W_g2, b_g2, eps, W_u, b_u, W_t1, b_t1, W_t2, b_t2, W_c1, b_c1, W_c2, b_c2, ln1_g, ln1_b, ln2_g, ln2_b, W_o1, b_o1, W_o2, b_o2):
    h = x @ W_in + b_in
    e_all = edge_attr @ W_edge + b_edge
    h = h[subgraphs_nodes_mapper]
    e = e_all[subgraphs_edges_mapper]
    src = combined_subgraphs[0]
    dst = combined_subgraphs[1]
    ones_sub = jnp.ones((N_SUB, 1), jnp.float32)
    for i in range(NLAYER_GNN):
        if i > 0:
            s = jax.ops.segment_sum(h, subgraphs_batch, num_segments=B * P)
            c = jax.ops.segment_sum(ones_sub, subgraphs_batch, num_segments=B * P)
            sub_mean = s / jnp.maximum(c, 1.0)
            u = jnp.maximum(sub_mean @ W_u + b_u, 0.0)
            h = h + u[subgraphs_batch]
            ns = jax.ops.segment_sum(h, subgraphs_nodes_mapper, num_segments=N)
            nc = jax.ops.segment_sum(ones_sub, subgraphs_nodes_mapper, num_segments=N)
            h = (ns / jnp.maximum(nc, 1.0))[subgraphs_nodes_mapper]
        msg = jnp.maximum(h[src] + e, 0.0)
        aggr = jax.ops.segment_sum(msg, dst, num_segments=N_SUB)
        z = (1.0 + eps[i]) * h + aggr
        z = jnp.maximum(z @ W_g[i] + b_g[i], 0.0)
        z = z @ W_g2[i] + b_g2[i]
        h = h + jnp.maximum(z, 0.0)
    s = jax.ops.segment_sum(h, subgraphs_batch, num_segments=B * P)
    c = jax.ops.segment_sum(ones_sub, subgraphs_batch, num_segments=B * P)
    subgraph_x = s / jnp.maximum(c, 1.0)
    mixer_x = subgraph_x.reshape(B, P, NHID)
    for i in range(NLAYER_MIX):
        y = _layernorm(mixer_x, ln1_g[i], ln1_b[i])
        y = jnp.swapaxes(y, 1, 2)
        y = jnp.maximum(y @ W_t1[i] + b_t1[i], 0.0) @ W_t2[i] + b_t2[i]
        y = jnp.swapaxes(y, 1, 2)
        mixer_x = mixer_x + y
        y = _layernorm(mixer_x, ln2_g[i], ln2_b[i])
        y = jnp.maximum(y @ W_c1[i] + b_c1[i], 0.0) @ W_c2[i] + b_c2[i]
        mixer_x = mixer_x + y
    pooled = (mixer_x * mask[..., None]).sum(1) / mask.sum(1, keepdims=True)
    out = jnp.maximum(pooled @ W_o1 + b_o1, 0.0) @ W_o2 + b_o2
    out = pl.pallas_call(
        _ident_kernel,
        out_shape=jax.ShapeDtypeStruct(out.shape, out.dtype),
    )(out)
    return out


# speedup vs baseline: 1.8228x; 1.8228x over previous
import functools

import jax
import jax.numpy as jnp
from jax import lax
from jax.experimental import pallas as pl
from jax.experimental.pallas import tpu as pltpu
from jax.experimental.pallas import tpu_sc as plsc

N = 10000
E = 160000
N_SUB = 20000
E_SUB = 320000
B = 4
P = 32
BP = B * P
NHID = 128
NFEAT_EDGE = 16
NLAYER_GNN = 2
NLAYER_MIX = 2

@functools.lru_cache(maxsize=1)
def _mesh():
    return plsc.VectorSubcoreMesh(core_axis_name="c", subcore_axis_name="s")


NC = 2
NS = 16
NW = NC * NS


def _sc_gather(table, idx, rows, chunk):
    V, D = table.shape
    assert rows % chunk == 0
    nchunks = rows // chunk

    def body(tab_ref, idx_ref, out_ref, idx_v, rows_v, sem):
        c = lax.axis_index("c")
        s = lax.axis_index("s")
        w = s * NC + c

        def one(i, _):
            ci = w + i * NW
            base = ci * chunk
            pltpu.sync_copy(idx_ref.at[pl.ds(base, chunk)], idx_v)
            pltpu.async_copy(tab_ref.at[idx_v], rows_v, sem).wait()
            pltpu.sync_copy(rows_v, out_ref.at[pl.ds(base, chunk)])
            return 0

        n_i = (nchunks - w + NW - 1) // NW
        lax.fori_loop(0, n_i, one, 0)

    fn = pl.kernel(
        body,
        mesh=_mesh(),
        out_type=jax.ShapeDtypeStruct((rows, D), jnp.float32),
        scratch_types=[
            pltpu.VMEM((chunk,), jnp.int32),
            pltpu.VMEM((chunk, D), jnp.float32),
            pltpu.SemaphoreType.DMA,
        ],
    )
    return fn(table, idx)


_CONV_CH = 128
_CONV_NCHUNK = E_SUB // _CONV_CH
_CONV_HALF = N_SUB // NC
_CONV_ACC = 10240


def _sc_conv(h, e_all, src, dst, emap):
    def body(h_ref, e_ref, src_ref, dst_ref, emap_ref, out_ref,
             idx_s, idx_d, idx_e, hrows, erows, acc, sem, sem2):
        c = lax.axis_index("c")
        s = lax.axis_index("s")
        lo = c * _CONV_HALF

        for v in range(8):
            hrows[0, pl.ds(v * 16, 16)] = jnp.zeros((16,), jnp.float32)

        def zrow(r, _):
            for v in range(8):
                hrows[r, pl.ds(v * 16, 16)] = jnp.zeros((16,), jnp.float32)
            return 0

        lax.fori_loop(1, 128, zrow, 0)
        for j in range(5):
            pltpu.sync_copy(hrows, acc.at[pl.ds(s * 640 + j * 128, 128)])
        plsc.subcore_barrier()

        def one(i, _):
            ci = s + i * NS
            base = ci * _CONV_CH
            pltpu.sync_copy(src_ref.at[pl.ds(base, _CONV_CH)], idx_s)
            pltpu.sync_copy(dst_ref.at[pl.ds(base, _CONV_CH)], idx_d)
            pltpu.sync_copy(emap_ref.at[pl.ds(base, _CONV_CH)], idx_e)
            cp_h = pltpu.async_copy(h_ref.at[idx_s], hrows, sem)
            cp_e = pltpu.async_copy(e_ref.at[idx_e], erows, sem2)
            cp_h.wait()
            cp_e.wait()

            def relu_row(r, _):
                for v in range(8):
                    hv = hrows[r, pl.ds(v * 16, 16)]
                    ev = erows[r, pl.ds(v * 16, 16)]
                    erows[r, pl.ds(v * 16, 16)] = jnp.maximum(hv + ev, 0.0)
                return 0

            lax.fori_loop(0, _CONV_CH, relu_row, 0)

            for k in range(_CONV_CH // 16):
                dv = idx_d[pl.ds(k * 16, 16)]
                m = (dv >= lo) & (dv < lo + _CONV_HALF)
                idx_d[pl.ds(k * 16, 16)] = jnp.where(
                    m, dv - lo, jnp.full((16,), _CONV_HALF, jnp.int32))
            pltpu.sync_copy(erows, acc.at[idx_d], add=True)
            return 0

        n_i = (_CONV_NCHUNK - s + NS - 1) // NS
        lax.fori_loop(0, n_i, one, 0)
        plsc.subcore_barrier()

        @pl.when(s < 15)
        def _():
            pltpu.sync_copy(acc.at[pl.ds(s * 640, 640)],
                            out_ref.at[pl.ds(lo + s * 640, 640)])

        @pl.when(s == 15)
        def _():
            pltpu.sync_copy(acc.at[pl.ds(9600, 400)],
                            out_ref.at[pl.ds(lo + 9600, 400)])

    fn = pl.kernel(
        body,
        mesh=_mesh(),
        out_type=jax.ShapeDtypeStruct((N_SUB, NHID), jnp.float32),
        scratch_types=[
            pltpu.VMEM((_CONV_CH,), jnp.int32),
            pltpu.VMEM((_CONV_CH,), jnp.int32),
            pltpu.VMEM((_CONV_CH,), jnp.int32),
            pltpu.VMEM((_CONV_CH, NHID), jnp.float32),
            pltpu.VMEM((_CONV_CH, NHID), jnp.float32),
            pltpu.VMEM_SHARED((_CONV_ACC, NHID), jnp.float32),
            pltpu.SemaphoreType.DMA,
            pltpu.SemaphoreType.DMA,
        ],
    )
    return fn(h, e_all, src, dst, emap)


_DD_CH = 80
_DD_NCHUNK = N_SUB // _DD_CH
_DD_HALF = N // NC
_DD_ACC = 5120


def _sc_dedup_mean(rows, mapper):
    def body(rows_ref, map_ref, out_ref,
             idx_d, rbuf, ones_v, cbuf, acc, cnt, sem):
        c = lax.axis_index("c")
        s = lax.axis_index("s")
        lo = c * _DD_HALF

        def zrow(r, _):
            for v in range(8):
                rbuf[r, pl.ds(v * 16, 16)] = jnp.zeros((16,), jnp.float32)
                ones_v[r, pl.ds(v * 16, 16)] = jnp.ones((16,), jnp.float32)
            return 0

        lax.fori_loop(0, _DD_CH, zrow, 0)
        for j in range(4):
            pltpu.sync_copy(rbuf, acc.at[pl.ds(s * 320 + j * 80, 80)])
            pltpu.sync_copy(rbuf, cnt.at[pl.ds(s * 320 + j * 80, 80)])
        plsc.subcore_barrier()

        def one(i, _):
            ci = s + i * NS
            base = ci * _DD_CH
            pltpu.sync_copy(map_ref.at[pl.ds(base, _DD_CH)], idx_d)
            pltpu.sync_copy(rows_ref.at[pl.ds(base, _DD_CH)], rbuf)
            for k in range(_DD_CH // 16):
                dv = idx_d[pl.ds(k * 16, 16)]
                m = (dv >= lo) & (dv < lo + _DD_HALF)
                idx_d[pl.ds(k * 16, 16)] = jnp.where(
                    m, dv - lo, jnp.full((16,), _DD_HALF, jnp.int32))
            pltpu.sync_copy(rbuf, acc.at[idx_d], add=True)
            pltpu.sync_copy(ones_v, cnt.at[idx_d], add=True)
            return 0

        n_i = (_DD_NCHUNK - s + NS - 1) // NS
        lax.fori_loop(0, n_i, one, 0)
        plsc.subcore_barrier()

        def flush(j, _):
            off = s * 320 + j * 40
            pltpu.sync_copy(acc.at[pl.ds(off, 40)], rbuf.at[pl.ds(0, 40)])
            pltpu.sync_copy(cnt.at[pl.ds(off, 40)], cbuf.at[pl.ds(0, 40)])

            def scale(r, _):
                inv = 1.0 / jnp.maximum(cbuf[r, pl.ds(0, 16)], 1.0)
                for v in range(8):
                    rbuf[r, pl.ds(v * 16, 16)] = rbuf[r, pl.ds(v * 16, 16)] * inv
                return 0

            lax.fori_loop(0, 40, scale, 0)
            pltpu.sync_copy(rbuf.at[pl.ds(0, 40)],
                            out_ref.at[pl.ds(lo + off, 40)])
            return 0

        n_f = jnp.where(s == 15, 5, 8)
        lax.fori_loop(0, n_f, flush, 0)

    fn = pl.kernel(
        body,
        mesh=_mesh(),
        out_type=jax.ShapeDtypeStruct((N, NHID), jnp.float32),
        scratch_types=[
            pltpu.VMEM((_DD_CH,), jnp.int32),
            pltpu.VMEM((_DD_CH, NHID), jnp.float32),
            pltpu.VMEM((_DD_CH, NHID), jnp.float32),
            pltpu.VMEM((_DD_CH, NHID), jnp.float32),
            pltpu.VMEM_SHARED((_DD_ACC, NHID), jnp.float32),
            pltpu.VMEM_SHARED((_DD_ACC, NHID), jnp.float32),
            pltpu.SemaphoreType.DMA,
        ],
    )
    return fn(rows, mapper)


def _tc_lin(x, W, b, blk):
    R, K = x.shape
    Kw, F = W.shape
    assert R % blk == 0

    def body(x_ref, w_ref, b_ref, o_ref):
        o_ref[...] = jax.lax.dot_general(
            x_ref[...], w_ref[...], (((1,), (0,)), ((), ())),
            preferred_element_type=jnp.float32) + b_ref[...]

    return pl.pallas_call(
        body,
        grid=(R // blk,),
        in_specs=[
            pl.BlockSpec((blk, K), lambda i: (i, 0)),
            pl.BlockSpec((Kw, F), lambda i: (0, 0)),
            pl.BlockSpec((1, F), lambda i: (0, 0)),
        ],
        out_specs=pl.BlockSpec((blk, F), lambda i: (i, 0)),
        out_shape=jax.ShapeDtypeStruct((R, F), jnp.float32),
    )(x, W, b.reshape(1, F))


_GNN_BLK = 400


def _tc_gnn_mlp(h, aggr, W1, b1, W2, b2, eps_i):
    def body(h_ref, a_ref, w1_ref, b1_ref, w2_ref, b2_ref, e_ref, o_ref):
        ep = e_ref[0, 0]
        z = (1.0 + ep) * h_ref[...] + a_ref[...]
        z = jax.lax.dot_general(z, w1_ref[...], (((1,), (0,)), ((), ())),
                                preferred_element_type=jnp.float32) + b1_ref[...]
        z = jnp.maximum(z, 0.0)
        z = jax.lax.dot_general(z, w2_ref[...], (((1,), (0,)), ((), ())),
                                preferred_element_type=jnp.float32) + b2_ref[...]
        o_ref[...] = h_ref[...] + jnp.maximum(z, 0.0)

    return pl.pallas_call(
        body,
        grid=(N_SUB // _GNN_BLK,),
        in_specs=[
            pl.BlockSpec((_GNN_BLK, NHID), lambda i: (i, 0)),
            pl.BlockSpec((_GNN_BLK, NHID), lambda i: (i, 0)),
            pl.BlockSpec((NHID, NHID), lambda i: (0, 0)),
            pl.BlockSpec((1, NHID), lambda i: (0, 0)),
            pl.BlockSpec((NHID, NHID), lambda i: (0, 0)),
            pl.BlockSpec((1, NHID), lambda i: (0, 0)),
            pl.BlockSpec((1, 1), lambda i: (0, 0)),
        ],
        out_specs=pl.BlockSpec((_GNN_BLK, NHID), lambda i: (i, 0)),
        out_shape=jax.ShapeDtypeStruct((N_SUB, NHID), jnp.float32),
    )(h, aggr, W1, b1.reshape(1, NHID), W2, b2.reshape(1, NHID),
      eps_i.reshape(1, 1))


def _tc_patch_pool(h, ids3, inv_c, W_u, b_u):
    nblk = N_SUB // _GNN_BLK

    def body(h_ref, id_ref, ic_ref, wu_ref, bu_ref, mean_ref, u_ref, s_acc):
        i = pl.program_id(0)

        @pl.when(i == 0)
        def _():
            s_acc[...] = jnp.zeros((BP, NHID), jnp.float32)

        ids = id_ref[0].reshape(_GNN_BLK, 1)
        onehot = (ids == jax.lax.broadcasted_iota(
            jnp.int32, (_GNN_BLK, BP), 1)).astype(jnp.float32)
        s_acc[...] += jax.lax.dot_general(
            onehot, h_ref[...], (((0,), (0,)), ((), ())),
            preferred_element_type=jnp.float32)

        @pl.when(i == nblk - 1)
        def _():
            mean = s_acc[...] * ic_ref[...]
            mean_ref[...] = mean
            u = jax.lax.dot_general(mean, wu_ref[...], (((1,), (0,)), ((), ())),
                                    preferred_element_type=jnp.float32)
            u_ref[...] = jnp.maximum(u + bu_ref[...], 0.0)

    return pl.pallas_call(
        body,
        grid=(nblk,),
        in_specs=[
            pl.BlockSpec((_GNN_BLK, NHID), lambda i: (i, 0)),
            pl.BlockSpec((1, 1, _GNN_BLK), lambda i: (i, 0, 0)),
            pl.BlockSpec((BP, 1), lambda i: (0, 0)),
            pl.BlockSpec((NHID, NHID), lambda i: (0, 0)),
            pl.BlockSpec((1, NHID), lambda i: (0, 0)),
        ],
        out_specs=[
            pl.BlockSpec((BP, NHID), lambda i: (0, 0)),
            pl.BlockSpec((BP, NHID), lambda i: (0, 0)),
        ],
        out_shape=[
            jax.ShapeDtypeStruct((BP, NHID), jnp.float32),
            jax.ShapeDtypeStruct((BP, NHID), jnp.float32),
        ],
        scratch_shapes=[pltpu.VMEM((BP, NHID), jnp.float32)],
    )(h, ids3, inv_c.reshape(BP, 1), W_u, b_u.reshape(1, NHID))


def _tc_broadcast_add(h, ids3, u):
    def body(h_ref, id_ref, u_ref, o_ref):
        ids = id_ref[0].reshape(_GNN_BLK, 1)
        onehot = (ids == jax.lax.broadcasted_iota(
            jnp.int32, (_GNN_BLK, BP), 1)).astype(jnp.float32)
        o_ref[...] = h_ref[...] + jax.lax.dot_general(
            onehot, u_ref[...], (((1,), (0,)), ((), ())),
            preferred_element_type=jnp.float32)

    return pl.pallas_call(
        body,
        grid=(N_SUB // _GNN_BLK,),
        in_specs=[
            pl.BlockSpec((_GNN_BLK, NHID), lambda i: (i, 0)),
            pl.BlockSpec((1, 1, _GNN_BLK), lambda i: (i, 0, 0)),
            pl.BlockSpec((BP, NHID), lambda i: (0, 0)),
        ],
        out_specs=pl.BlockSpec((_GNN_BLK, NHID), lambda i: (i, 0)),
        out_shape=jax.ShapeDtypeStruct((N_SUB, NHID), jnp.float32),
    )(h, ids3, u)


def _tc_mixer_head(sub_mean, mask_row,
                   W_t1, b_t1, W_t2, b_t2, W_c1, b_c1, W_c2, b_c2,
                   ln1_g, ln1_b, ln2_g, ln2_b, W_o1, b_o1, W_o2, b_o2):
    def ln(x, g, b):
        m = jnp.mean(x, axis=-1, keepdims=True)
        v = jnp.mean((x - m) ** 2, axis=-1, keepdims=True)
        return (x - m) / jnp.sqrt(v + 1e-5) * g + b

    def mm(a, bmat):
        return jax.lax.dot_general(a, bmat, (((1,), (0,)), ((), ())),
                                   preferred_element_type=jnp.float32)

    def body(x_ref, mk_ref, wt1_ref, bt1_ref, wt2_ref, bt2_ref,
             wc1_ref, bc1_ref, wc2_ref, bc2_ref,
             l1g_ref, l1b_ref, l2g_ref, l2b_ref,
             wo1_ref, bo1_ref, wo2_ref, bo2_ref, o_ref):
        xcur = x_ref[...]
        for i in range(NLAYER_MIX):
            y = ln(xcur, l1g_ref[i], l1b_ref[i])
            parts = []
            for bi in range(B):
                yb = y[bi * P:(bi + 1) * P, :]
                z = yb.T
                z = jnp.maximum(mm(z, wt1_ref[i]) + bt1_ref[i], 0.0)
                z = mm(z, wt2_ref[i]) + bt2_ref[i]
                parts.append(z.T)
            xcur = xcur + jnp.concatenate(parts, axis=0)
            y = ln(xcur, l2g_ref[i], l2b_ref[i])
            y = jnp.maximum(mm(y, wc1_ref[i]) + bc1_ref[i], 0.0)
            xcur = xcur + mm(y, wc2_ref[i]) + bc2_ref[i]
        mk = mk_ref[...].reshape(BP, 1)
        w = xcur * mk
        pooled = []
        for bi in range(B):
            seg = w[bi * P:(bi + 1) * P, :]
            den = jnp.sum(mk[bi * P:(bi + 1) * P, :])
            pooled.append(jnp.sum(seg, axis=0, keepdims=True) /
                          jnp.maximum(den, 1e-9))
        pooled = jnp.concatenate(pooled, axis=0)
        z = jnp.maximum(mm(pooled, wo1_ref[...]) + bo1_ref[...], 0.0)
        out = mm(z, wo2_ref[...]) + bo2_ref[...]
        o_ref[...] = jnp.pad(out, ((0, 4), (0, 64)))

    args = (sub_mean, mask_row, W_t1, b_t1, W_t2, b_t2, W_c1, b_c1, W_c2,
            b_c2, ln1_g, ln1_b, ln2_g, ln2_b, W_o1, b_o1.reshape(1, NHID),
            W_o2, b_o2.reshape(1, 64))
    return pl.pallas_call(
        body,
        out_shape=jax.ShapeDtypeStruct((8, 128), jnp.float32),
    )(*args)


def kernel(x, edge_attr, combined_subgraphs, subgraphs_nodes_mapper,
           subgraphs_edges_mapper, subgraphs_batch, mask, W_in, b_in,
           W_edge, b_edge, W_g, b_g, W_g2, b_g2, eps, W_u, b_u,
           W_t1, b_t1, W_t2, b_t2, W_c1, b_c1, W_c2, b_c2,
           ln1_g, ln1_b, ln2_g, ln2_b, W_o1, b_o1, W_o2, b_o2):
    src = combined_subgraphs[0].astype(jnp.int32)
    dst = combined_subgraphs[1].astype(jnp.int32)
    nmap = subgraphs_nodes_mapper.astype(jnp.int32)
    emap = subgraphs_edges_mapper.astype(jnp.int32)
    batch = subgraphs_batch.astype(jnp.int32)

    bounds = jnp.searchsorted(batch, jnp.arange(BP + 1, dtype=jnp.int32))
    c = (bounds[1:] - bounds[:-1]).astype(jnp.float32)
    inv_c = 1.0 / jnp.maximum(c, 1.0)
    ids3 = batch.reshape(N_SUB // _GNN_BLK, 1, _GNN_BLK)

    h0 = _tc_lin(x, W_in, b_in, blk=400)
    h = _sc_gather(h0, nmap, N_SUB, chunk=80)
    e_all = _tc_lin(edge_attr, W_edge, b_edge, blk=640)

    for i in range(NLAYER_GNN):
        if i > 0:
            _, u = _tc_patch_pool(h, ids3, inv_c, W_u, b_u)
            h = _tc_broadcast_add(h, ids3, u)
            means = _sc_dedup_mean(h, nmap)
            h = _sc_gather(means, nmap, N_SUB, chunk=80)
        aggr = _sc_conv(h, e_all, src, dst, emap)
        h = _tc_gnn_mlp(h, aggr, W_g[i], b_g[i], W_g2[i], b_g2[i], eps[i])

    sub_mean, _ = _tc_patch_pool(h, ids3, inv_c, W_u, b_u)
    out_pad = _tc_mixer_head(
        sub_mean, mask.reshape(1, BP), W_t1, b_t1, W_t2, b_t2,
        W_c1, b_c1, W_c2, b_c2, ln1_g, ln1_b, ln2_g, ln2_b,
        W_o1, b_o1, W_o2, b_o2)
    return out_pad[:B, :64]


# speedup vs baseline: 2.6192x; 1.4369x over previous
import functools

import jax
import jax.numpy as jnp
from jax import lax
from jax.experimental import pallas as pl
from jax.experimental.pallas import tpu as pltpu
from jax.experimental.pallas import tpu_sc as plsc

N = 10000
E = 160000
N_SUB = 20000
E_SUB = 320000
B = 4
P = 32
BP = B * P
NHID = 128
NFEAT_EDGE = 16
NLAYER_GNN = 2
NLAYER_MIX = 2

@functools.lru_cache(maxsize=1)
def _mesh():
    return plsc.VectorSubcoreMesh(core_axis_name="c", subcore_axis_name="s")


NC = 2
NS = 16
NW = NC * NS


def _sc_gather(table, idx, rows, chunk):
    V, D = table.shape
    assert rows % chunk == 0
    nchunks = rows // chunk

    def body(tab_ref, idx_ref, out_ref, idx_v, rows_v, sem):
        c = lax.axis_index("c")
        s = lax.axis_index("s")
        w = s * NC + c

        def one(i, _):
            ci = w + i * NW
            base = ci * chunk
            pltpu.sync_copy(idx_ref.at[pl.ds(base, chunk)], idx_v)
            pltpu.async_copy(tab_ref.at[idx_v], rows_v, sem).wait()
            pltpu.sync_copy(rows_v, out_ref.at[pl.ds(base, chunk)])
            return 0

        n_i = (nchunks - w + NW - 1) // NW
        lax.fori_loop(0, n_i, one, 0)

    fn = pl.kernel(
        body,
        mesh=_mesh(),
        out_type=jax.ShapeDtypeStruct((rows, D), jnp.float32),
        scratch_types=[
            pltpu.VMEM((chunk,), jnp.int32),
            pltpu.VMEM((chunk, D), jnp.float32),
            pltpu.SemaphoreType.DMA,
        ],
    )
    return fn(table, idx)


_CONV_CH = 64
_CONV_NCHUNK = E_SUB // _CONV_CH
_CONV_HALF = N_SUB // NC
_CONV_ACC = 10240


def _sc_conv(h, e_all, idx_all):
    def body(h_ref, e_ref, ix_ref, out_ref,
             ix0, ix1, hr0, hr1, er0, er1, acc, sh0, sh1, se0, se1):
        c = lax.axis_index("c")
        s = lax.axis_index("s")
        lo = c * _CONV_HALF
        ixs = (ix0, ix1)
        hrs = (hr0, hr1)
        ers = (er0, er1)
        shs = (sh0, sh1)
        ses = (se0, se1)

        def zrow(r, _):
            for v in range(8):
                hr0[r, pl.ds(v * 16, 16)] = jnp.zeros((16,), jnp.float32)
            return 0

        lax.fori_loop(0, _CONV_CH, zrow, 0)
        for j in range(640 // _CONV_CH):
            pltpu.sync_copy(hr0, acc.at[pl.ds(s * 640 + j * _CONV_CH, _CONV_CH)])
        plsc.subcore_barrier()

        n_i = (_CONV_NCHUNK - s + NS - 1) // NS

        def prefetch(g, b):
            @pl.when(g < n_i)
            def _():
                ci = s + g * NS
                pltpu.sync_copy(ix_ref.at[ci], ixs[b])
                pltpu.async_copy(h_ref.at[ixs[b].at[0]], hrs[b], shs[b])
                pltpu.async_copy(e_ref.at[ixs[b].at[2]], ers[b], ses[b])

        def process(g, b):
            @pl.when(g < n_i)
            def _():
                hr, er, ix = hrs[b], ers[b], ixs[b]
                pltpu.make_async_copy(h_ref.at[ix.at[0]], hr, shs[b]).wait()
                pltpu.make_async_copy(e_ref.at[ix.at[2]], er, ses[b]).wait()

                def relu_row(r, _):
                    for v in range(8):
                        hv = hr[r, pl.ds(v * 16, 16)]
                        ev = er[r, pl.ds(v * 16, 16)]
                        er[r, pl.ds(v * 16, 16)] = jnp.maximum(hv + ev, 0.0)
                    return 0

                lax.fori_loop(0, _CONV_CH, relu_row, 0)

                for k in range(_CONV_CH // 16):
                    dv = ix[1, pl.ds(k * 16, 16)]
                    m = (dv >= lo) & (dv < lo + _CONV_HALF)
                    ix[1, pl.ds(k * 16, 16)] = jnp.where(
                        m, dv - lo, jnp.full((16,), _CONV_HALF, jnp.int32))
                pltpu.sync_copy(er, acc.at[ix.at[1]], add=True)

        prefetch(0, 0)

        def pair(j, _):
            g0 = 2 * j
            prefetch(g0 + 1, 1)
            process(g0, 0)
            prefetch(g0 + 2, 0)
            process(g0 + 1, 1)
            return 0

        lax.fori_loop(0, (n_i + 1) // 2, pair, 0)
        plsc.subcore_barrier()

        @pl.when(s < 15)
        def _():
            pltpu.sync_copy(acc.at[pl.ds(s * 640, 640)],
                            out_ref.at[pl.ds(lo + s * 640, 640)])

        @pl.when(s == 15)
        def _():
            pltpu.sync_copy(acc.at[pl.ds(9600, 400)],
                            out_ref.at[pl.ds(lo + 9600, 400)])

    fn = pl.kernel(
        body,
        mesh=_mesh(),
        out_type=jax.ShapeDtypeStruct((N_SUB, NHID), jnp.float32),
        scratch_types=[
            pltpu.VMEM((3, _CONV_CH), jnp.int32),
            pltpu.VMEM((3, _CONV_CH), jnp.int32),
            pltpu.VMEM((_CONV_CH, NHID), jnp.float32),
            pltpu.VMEM((_CONV_CH, NHID), jnp.float32),
            pltpu.VMEM((_CONV_CH, NHID), jnp.float32),
            pltpu.VMEM((_CONV_CH, NHID), jnp.float32),
            pltpu.VMEM_SHARED((_CONV_ACC, NHID), jnp.float32),
            pltpu.SemaphoreType.DMA,
            pltpu.SemaphoreType.DMA,
            pltpu.SemaphoreType.DMA,
            pltpu.SemaphoreType.DMA,
        ],
    )
    return fn(h, e_all, idx_all)


_DD_CH = 80
_DD_NCHUNK = N_SUB // _DD_CH
_DD_HALF = N // NC
_DD_ACC = 5120


def _sc_dedup_mean(rows, mapper):
    def body(rows_ref, map_ref, out_ref,
             idx_d, rbuf, ones_v, cbuf, acc, cnt, sem):
        c = lax.axis_index("c")
        s = lax.axis_index("s")
        lo = c * _DD_HALF

        def zrow(r, _):
            for v in range(8):
                rbuf[r, pl.ds(v * 16, 16)] = jnp.zeros((16,), jnp.float32)
                ones_v[r, pl.ds(v * 16, 16)] = jnp.ones((16,), jnp.float32)
            return 0

        lax.fori_loop(0, _DD_CH, zrow, 0)
        for j in range(4):
            pltpu.sync_copy(rbuf, acc.at[pl.ds(s * 320 + j * 80, 80)])
            pltpu.sync_copy(rbuf, cnt.at[pl.ds(s * 320 + j * 80, 80)])
        plsc.subcore_barrier()

        def one(i, _):
            ci = s + i * NS
            base = ci * _DD_CH
            pltpu.sync_copy(map_ref.at[pl.ds(base, _DD_CH)], idx_d)
            pltpu.sync_copy(rows_ref.at[pl.ds(base, _DD_CH)], rbuf)
            for k in range(_DD_CH // 16):
                dv = idx_d[pl.ds(k * 16, 16)]
                m = (dv >= lo) & (dv < lo + _DD_HALF)
                idx_d[pl.ds(k * 16, 16)] = jnp.where(
                    m, dv - lo, jnp.full((16,), _DD_HALF, jnp.int32))
            pltpu.sync_copy(rbuf, acc.at[idx_d], add=True)
            pltpu.sync_copy(ones_v, cnt.at[idx_d], add=True)
            return 0

        n_i = (_DD_NCHUNK - s + NS - 1) // NS
        lax.fori_loop(0, n_i, one, 0)
        plsc.subcore_barrier()

        def flush(j, _):
            off = s * 320 + j * 40
            pltpu.sync_copy(acc.at[pl.ds(off, 40)], rbuf.at[pl.ds(0, 40)])
            pltpu.sync_copy(cnt.at[pl.ds(off, 40)], cbuf.at[pl.ds(0, 40)])

            def scale(r, _):
                inv = 1.0 / jnp.maximum(cbuf[r, pl.ds(0, 16)], 1.0)
                for v in range(8):
                    rbuf[r, pl.ds(v * 16, 16)] = rbuf[r, pl.ds(v * 16, 16)] * inv
                return 0

            lax.fori_loop(0, 40, scale, 0)
            pltpu.sync_copy(rbuf.at[pl.ds(0, 40)],
                            out_ref.at[pl.ds(lo + off, 40)])
            return 0

        n_f = jnp.where(s == 15, 5, 8)
        lax.fori_loop(0, n_f, flush, 0)

    fn = pl.kernel(
        body,
        mesh=_mesh(),
        out_type=jax.ShapeDtypeStruct((N, NHID), jnp.float32),
        scratch_types=[
            pltpu.VMEM((_DD_CH,), jnp.int32),
            pltpu.VMEM((_DD_CH, NHID), jnp.float32),
            pltpu.VMEM((_DD_CH, NHID), jnp.float32),
            pltpu.VMEM((_DD_CH, NHID), jnp.float32),
            pltpu.VMEM_SHARED((_DD_ACC, NHID), jnp.float32),
            pltpu.VMEM_SHARED((_DD_ACC, NHID), jnp.float32),
            pltpu.SemaphoreType.DMA,
        ],
    )
    return fn(rows, mapper)


def _tc_lin(x, W, b, blk):
    R, K = x.shape
    Kw, F = W.shape
    assert R % blk == 0

    def body(x_ref, w_ref, b_ref, o_ref):
        o_ref[...] = jax.lax.dot_general(
            x_ref[...], w_ref[...], (((1,), (0,)), ((), ())),
            preferred_element_type=jnp.float32) + b_ref[...]

    return pl.pallas_call(
        body,
        grid=(R // blk,),
        in_specs=[
            pl.BlockSpec((blk, K), lambda i: (i, 0)),
            pl.BlockSpec((Kw, F), lambda i: (0, 0)),
            pl.BlockSpec((1, F), lambda i: (0, 0)),
        ],
        out_specs=pl.BlockSpec((blk, F), lambda i: (i, 0)),
        out_shape=jax.ShapeDtypeStruct((R, F), jnp.float32),
    )(x, W, b.reshape(1, F))


_GNN_BLK = 400


def _tc_gnn_mlp(h, aggr, W1, b1, W2, b2, eps_i):
    def body(h_ref, a_ref, w1_ref, b1_ref, w2_ref, b2_ref, e_ref, o_ref):
        ep = e_ref[0, 0]
        z = (1.0 + ep) * h_ref[...] + a_ref[...]
        z = jax.lax.dot_general(z, w1_ref[...], (((1,), (0,)), ((), ())),
                                preferred_element_type=jnp.float32) + b1_ref[...]
        z = jnp.maximum(z, 0.0)
        z = jax.lax.dot_general(z, w2_ref[...], (((1,), (0,)), ((), ())),
                                preferred_element_type=jnp.float32) + b2_ref[...]
        o_ref[...] = h_ref[...] + jnp.maximum(z, 0.0)

    return pl.pallas_call(
        body,
        grid=(N_SUB // _GNN_BLK,),
        in_specs=[
            pl.BlockSpec((_GNN_BLK, NHID), lambda i: (i, 0)),
            pl.BlockSpec((_GNN_BLK, NHID), lambda i: (i, 0)),
            pl.BlockSpec((NHID, NHID), lambda i: (0, 0)),
            pl.BlockSpec((1, NHID), lambda i: (0, 0)),
            pl.BlockSpec((NHID, NHID), lambda i: (0, 0)),
            pl.BlockSpec((1, NHID), lambda i: (0, 0)),
            pl.BlockSpec((1, 1), lambda i: (0, 0)),
        ],
        out_specs=pl.BlockSpec((_GNN_BLK, NHID), lambda i: (i, 0)),
        out_shape=jax.ShapeDtypeStruct((N_SUB, NHID), jnp.float32),
    )(h, aggr, W1, b1.reshape(1, NHID), W2, b2.reshape(1, NHID),
      eps_i.reshape(1, 1))


def _tc_patch_pool(h, ids3, inv_c, W_u, b_u):
    nblk = N_SUB // _GNN_BLK

    def body(h_ref, id_ref, ic_ref, wu_ref, bu_ref, mean_ref, u_ref, s_acc):
        i = pl.program_id(0)

        @pl.when(i == 0)
        def _():
            s_acc[...] = jnp.zeros((BP, NHID), jnp.float32)

        ids = id_ref[0].reshape(_GNN_BLK, 1)
        onehot = (ids == jax.lax.broadcasted_iota(
            jnp.int32, (_GNN_BLK, BP), 1)).astype(jnp.float32)
        s_acc[...] += jax.lax.dot_general(
            onehot, h_ref[...], (((0,), (0,)), ((), ())),
            preferred_element_type=jnp.float32)

        @pl.when(i == nblk - 1)
        def _():
            mean = s_acc[...] * ic_ref[...]
            mean_ref[...] = mean
            u = jax.lax.dot_general(mean, wu_ref[...], (((1,), (0,)), ((), ())),
                                    preferred_element_type=jnp.float32)
            u_ref[...] = jnp.maximum(u + bu_ref[...], 0.0)

    return pl.pallas_call(
        body,
        grid=(nblk,),
        in_specs=[
            pl.BlockSpec((_GNN_BLK, NHID), lambda i: (i, 0)),
            pl.BlockSpec((1, 1, _GNN_BLK), lambda i: (i, 0, 0)),
            pl.BlockSpec((BP, 1), lambda i: (0, 0)),
            pl.BlockSpec((NHID, NHID), lambda i: (0, 0)),
            pl.BlockSpec((1, NHID), lambda i: (0, 0)),
        ],
        out_specs=[
            pl.BlockSpec((BP, NHID), lambda i: (0, 0)),
            pl.BlockSpec((BP, NHID), lambda i: (0, 0)),
        ],
        out_shape=[
            jax.ShapeDtypeStruct((BP, NHID), jnp.float32),
            jax.ShapeDtypeStruct((BP, NHID), jnp.float32),
        ],
        scratch_shapes=[pltpu.VMEM((BP, NHID), jnp.float32)],
    )(h, ids3, inv_c.reshape(BP, 1), W_u, b_u.reshape(1, NHID))


def _tc_broadcast_add(h, ids3, u):
    def body(h_ref, id_ref, u_ref, o_ref):
        ids = id_ref[0].reshape(_GNN_BLK, 1)
        onehot = (ids == jax.lax.broadcasted_iota(
            jnp.int32, (_GNN_BLK, BP), 1)).astype(jnp.float32)
        o_ref[...] = h_ref[...] + jax.lax.dot_general(
            onehot, u_ref[...], (((1,), (0,)), ((), ())),
            preferred_element_type=jnp.float32)

    return pl.pallas_call(
        body,
        grid=(N_SUB // _GNN_BLK,),
        in_specs=[
            pl.BlockSpec((_GNN_BLK, NHID), lambda i: (i, 0)),
            pl.BlockSpec((1, 1, _GNN_BLK), lambda i: (i, 0, 0)),
            pl.BlockSpec((BP, NHID), lambda i: (0, 0)),
        ],
        out_specs=pl.BlockSpec((_GNN_BLK, NHID), lambda i: (i, 0)),
        out_shape=jax.ShapeDtypeStruct((N_SUB, NHID), jnp.float32),
    )(h, ids3, u)


def _tc_mixer_head(sub_mean, mask_row,
                   W_t1, b_t1, W_t2, b_t2, W_c1, b_c1, W_c2, b_c2,
                   ln1_g, ln1_b, ln2_g, ln2_b, W_o1, b_o1, W_o2, b_o2):
    def ln(x, g, b):
        m = jnp.mean(x, axis=-1, keepdims=True)
        v = jnp.mean((x - m) ** 2, axis=-1, keepdims=True)
        return (x - m) / jnp.sqrt(v + 1e-5) * g + b

    def mm(a, bmat):
        return jax.lax.dot_general(a, bmat, (((1,), (0,)), ((), ())),
                                   preferred_element_type=jnp.float32)

    def body(x_ref, mk_ref, wt1_ref, bt1_ref, wt2_ref, bt2_ref,
             wc1_ref, bc1_ref, wc2_ref, bc2_ref,
             l1g_ref, l1b_ref, l2g_ref, l2b_ref,
             wo1_ref, bo1_ref, wo2_ref, bo2_ref, o_ref):
        xcur = x_ref[...]
        for i in range(NLAYER_MIX):
            y = ln(xcur, l1g_ref[i], l1b_ref[i])
            parts = []
            for bi in range(B):
                yb = y[bi * P:(bi + 1) * P, :]
                z = yb.T
                z = jnp.maximum(mm(z, wt1_ref[i]) + bt1_ref[i], 0.0)
                z = mm(z, wt2_ref[i]) + bt2_ref[i]
                parts.append(z.T)
            xcur = xcur + jnp.concatenate(parts, axis=0)
            y = ln(xcur, l2g_ref[i], l2b_ref[i])
            y = jnp.maximum(mm(y, wc1_ref[i]) + bc1_ref[i], 0.0)
            xcur = xcur + mm(y, wc2_ref[i]) + bc2_ref[i]
        mk = mk_ref[...].reshape(BP, 1)
        w = xcur * mk
        pooled = []
        for bi in range(B):
            seg = w[bi * P:(bi + 1) * P, :]
            den = jnp.sum(mk[bi * P:(bi + 1) * P, :])
            pooled.append(jnp.sum(seg, axis=0, keepdims=True) /
                          jnp.maximum(den, 1e-9))
        pooled = jnp.concatenate(pooled, axis=0)
        z = jnp.maximum(mm(pooled, wo1_ref[...]) + bo1_ref[...], 0.0)
        out = mm(z, wo2_ref[...]) + bo2_ref[...]
        o_ref[...] = jnp.pad(out, ((0, 4), (0, 64)))

    args = (sub_mean, mask_row, W_t1, b_t1, W_t2, b_t2, W_c1, b_c1, W_c2,
            b_c2, ln1_g, ln1_b, ln2_g, ln2_b, W_o1, b_o1.reshape(1, NHID),
            W_o2, b_o2.reshape(1, 64))
    return pl.pallas_call(
        body,
        out_shape=jax.ShapeDtypeStruct((8, 128), jnp.float32),
    )(*args)


def kernel(x, edge_attr, combined_subgraphs, subgraphs_nodes_mapper,
           subgraphs_edges_mapper, subgraphs_batch, mask, W_in, b_in,
           W_edge, b_edge, W_g, b_g, W_g2, b_g2, eps, W_u, b_u,
           W_t1, b_t1, W_t2, b_t2, W_c1, b_c1, W_c2, b_c2,
           ln1_g, ln1_b, ln2_g, ln2_b, W_o1, b_o1, W_o2, b_o2):
    src = combined_subgraphs[0].astype(jnp.int32)
    dst = combined_subgraphs[1].astype(jnp.int32)
    nmap = subgraphs_nodes_mapper.astype(jnp.int32)
    emap = subgraphs_edges_mapper.astype(jnp.int32)
    batch = subgraphs_batch.astype(jnp.int32)

    idx_all = jnp.stack([src.reshape(_CONV_NCHUNK, _CONV_CH),
                         dst.reshape(_CONV_NCHUNK, _CONV_CH),
                         emap.reshape(_CONV_NCHUNK, _CONV_CH)], axis=1)
    bounds = jnp.searchsorted(batch, jnp.arange(BP + 1, dtype=jnp.int32))
    c = (bounds[1:] - bounds[:-1]).astype(jnp.float32)
    inv_c = 1.0 / jnp.maximum(c, 1.0)
    ids3 = batch.reshape(N_SUB // _GNN_BLK, 1, _GNN_BLK)

    h0 = _tc_lin(x, W_in, b_in, blk=400)
    h = _sc_gather(h0, nmap, N_SUB, chunk=80)
    e_all = _tc_lin(edge_attr, W_edge, b_edge, blk=640)

    for i in range(NLAYER_GNN):
        if i > 0:
            _, u = _tc_patch_pool(h, ids3, inv_c, W_u, b_u)
            h = _tc_broadcast_add(h, ids3, u)
            means = _sc_dedup_mean(h, nmap)
            h = _sc_gather(means, nmap, N_SUB, chunk=80)
        aggr = _sc_conv(h, e_all, idx_all)
        h = _tc_gnn_mlp(h, aggr, W_g[i], b_g[i], W_g2[i], b_g2[i], eps[i])

    sub_mean, _ = _tc_patch_pool(h, ids3, inv_c, W_u, b_u)
    out_pad = _tc_mixer_head(
        sub_mean, mask.reshape(1, BP), W_t1, b_t1, W_t2, b_t2,
        W_c1, b_c1, W_c2, b_c2, ln1_g, ln1_b, ln2_g, ln2_b,
        W_o1, b_o1, W_o2, b_o2)
    return out_pad[:B, :64]


# speedup vs baseline: 2.7496x; 1.0498x over previous
import functools

import jax
import jax.numpy as jnp
from jax import lax
from jax.experimental import pallas as pl
from jax.experimental.pallas import tpu as pltpu
from jax.experimental.pallas import tpu_sc as plsc

N = 10000
E = 160000
N_SUB = 20000
E_SUB = 320000
B = 4
P = 32
BP = B * P
NHID = 128
NFEAT_EDGE = 16
NLAYER_GNN = 2
NLAYER_MIX = 2

@functools.lru_cache(maxsize=1)
def _mesh():
    return plsc.VectorSubcoreMesh(core_axis_name="c", subcore_axis_name="s")


NC = 2
NS = 16
NW = NC * NS


def _sc_gather(table, idx, rows, chunk):
    V, D = table.shape
    assert rows % chunk == 0
    nchunks = rows // chunk

    def body(tab_ref, idx_ref, out_ref, idx_v, rows_v, sem):
        c = lax.axis_index("c")
        s = lax.axis_index("s")
        w = s * NC + c

        def one(i, _):
            ci = w + i * NW
            base = ci * chunk
            pltpu.sync_copy(idx_ref.at[pl.ds(base, chunk)], idx_v)
            pltpu.async_copy(tab_ref.at[idx_v], rows_v, sem).wait()
            pltpu.sync_copy(rows_v, out_ref.at[pl.ds(base, chunk)])
            return 0

        n_i = (nchunks - w + NW - 1) // NW
        lax.fori_loop(0, n_i, one, 0)

    fn = pl.kernel(
        body,
        mesh=_mesh(),
        out_type=jax.ShapeDtypeStruct((rows, D), jnp.float32),
        scratch_types=[
            pltpu.VMEM((chunk,), jnp.int32),
            pltpu.VMEM((chunk, D), jnp.float32),
            pltpu.SemaphoreType.DMA,
        ],
    )
    return fn(table, idx)


_CONV_CH = 64
_CONV_NCHUNK = E_SUB // _CONV_CH
_CONV_HALF = N_SUB // NC
_CONV_ACC = 10240


def _sc_conv(h, e_all, idx_all):
    def body(h_ref, e_ref, ix_ref, out_ref,
             ix0, ix1, sx0, sx1, hr0, hr1, er0, er1, acc,
             si0, si1, sh0, sh1, se0, se1, sc0, sc1):
        c = lax.axis_index("c")
        s = lax.axis_index("s")
        lo = c * _CONV_HALF
        ixs = (ix0, ix1)
        sxs = (sx0, sx1)
        hrs = (hr0, hr1)
        ers = (er0, er1)
        sis = (si0, si1)
        shs = (sh0, sh1)
        ses = (se0, se1)
        scs = (sc0, sc1)

        def zrow(r, _):
            for v in range(8):
                hr0[r, pl.ds(v * 16, 16)] = jnp.zeros((16,), jnp.float32)
            return 0

        lax.fori_loop(0, _CONV_CH, zrow, 0)
        for j in range(640 // _CONV_CH):
            pltpu.sync_copy(hr0, acc.at[pl.ds(s * 640 + j * _CONV_CH, _CONV_CH)])
        plsc.subcore_barrier()

        n_i = (_CONV_NCHUNK - s + NS - 1) // NS

        def fire_idx(g, b):
            @pl.when(g < n_i)
            def _():
                pltpu.async_copy(ix_ref.at[s + g * NS], ixs[b], sis[b])

        def fire_gathers(g, b):
            @pl.when(g < n_i)
            def _():
                pltpu.make_async_copy(ix_ref.at[s + g * NS], ixs[b], sis[b]).wait()
                pltpu.async_copy(h_ref.at[ixs[b].at[0]], hrs[b], shs[b])
                pltpu.async_copy(e_ref.at[ixs[b].at[2]], ers[b], ses[b])

        def step(g, b):
            @pl.when(g < n_i)
            def _():
                ix, sx, hr, er = ixs[b], sxs[b], hrs[b], ers[b]
                pltpu.make_async_copy(h_ref.at[ix.at[0]], hr, shs[b]).wait()
                pltpu.make_async_copy(e_ref.at[ix.at[2]], er, ses[b]).wait()
                for k in range(_CONV_CH // 16):
                    dv = ix[1, pl.ds(k * 16, 16)]
                    m = (dv >= lo) & (dv < lo + _CONV_HALF)
                    sx[pl.ds(k * 16, 16)] = jnp.where(
                        m, dv - lo, jnp.full((16,), _CONV_HALF, jnp.int32))
                fire_idx(g + 2, b)
                @pl.when(g >= 1)
                def _():
                    pltpu.make_async_copy(
                        ers[1 - b], acc.at[sxs[1 - b]], scs[1 - b]).wait()
                fire_gathers(g + 1, 1 - b)
                def relu_row(r, _):
                    for v in range(8):
                        hv = hr[r, pl.ds(v * 16, 16)]
                        ev = er[r, pl.ds(v * 16, 16)]
                        er[r, pl.ds(v * 16, 16)] = jnp.maximum(hv + ev, 0.0)
                    return 0

                lax.fori_loop(0, _CONV_CH, relu_row, 0)
                pltpu.async_copy(er, acc.at[sx], scs[b], add=True)

        fire_idx(0, 0)
        fire_idx(1, 1)
        fire_gathers(0, 0)

        def pair(j, _):
            step(2 * j, 0)
            step(2 * j + 1, 1)
            return 0

        lax.fori_loop(0, (n_i + 1) // 2, pair, 0)

        @pl.when((n_i % 2) == 1)
        def _():
            pltpu.make_async_copy(er0, acc.at[sx0], sc0).wait()

        @pl.when((n_i % 2) == 0)
        def _():
            pltpu.make_async_copy(er1, acc.at[sx1], sc1).wait()

        plsc.subcore_barrier()

        @pl.when(s < 15)
        def _():
            pltpu.sync_copy(acc.at[pl.ds(s * 640, 640)],
                            out_ref.at[pl.ds(lo + s * 640, 640)])

        @pl.when(s == 15)
        def _():
            pltpu.sync_copy(acc.at[pl.ds(9600, 400)],
                            out_ref.at[pl.ds(lo + 9600, 400)])

    fn = pl.kernel(
        body,
        mesh=_mesh(),
        out_type=jax.ShapeDtypeStruct((N_SUB, NHID), jnp.float32),
        scratch_types=[
            pltpu.VMEM((3, _CONV_CH), jnp.int32),
            pltpu.VMEM((3, _CONV_CH), jnp.int32),
            pltpu.VMEM((_CONV_CH,), jnp.int32),
            pltpu.VMEM((_CONV_CH,), jnp.int32),
            pltpu.VMEM((_CONV_CH, NHID), jnp.float32),
            pltpu.VMEM((_CONV_CH, NHID), jnp.float32),
            pltpu.VMEM((_CONV_CH, NHID), jnp.float32),
            pltpu.VMEM((_CONV_CH, NHID), jnp.float32),
            pltpu.VMEM_SHARED((_CONV_ACC, NHID), jnp.float32),
            pltpu.SemaphoreType.DMA,
            pltpu.SemaphoreType.DMA,
            pltpu.SemaphoreType.DMA,
            pltpu.SemaphoreType.DMA,
            pltpu.SemaphoreType.DMA,
            pltpu.SemaphoreType.DMA,
            pltpu.SemaphoreType.DMA,
            pltpu.SemaphoreType.DMA,
        ],
    )
    return fn(h, e_all, idx_all)


_DD_CH = 80
_DD_NCHUNK = N_SUB // _DD_CH
_DD_HALF = N // NC
_DD_ACC = 5120


def _sc_dedup_mean(rows, mapper):
    def body(rows_ref, map_ref, out_ref,
             idx_d, rbuf, ones_v, cbuf, acc, cnt, sem):
        c = lax.axis_index("c")
        s = lax.axis_index("s")
        lo = c * _DD_HALF

        def zrow(r, _):
            for v in range(8):
                rbuf[r, pl.ds(v * 16, 16)] = jnp.zeros((16,), jnp.float32)
                ones_v[r, pl.ds(v * 16, 16)] = jnp.ones((16,), jnp.float32)
            return 0

        lax.fori_loop(0, _DD_CH, zrow, 0)
        for j in range(4):
            pltpu.sync_copy(rbuf, acc.at[pl.ds(s * 320 + j * 80, 80)])
            pltpu.sync_copy(rbuf, cnt.at[pl.ds(s * 320 + j * 80, 80)])
        plsc.subcore_barrier()

        def one(i, _):
            ci = s + i * NS
            base = ci * _DD_CH
            pltpu.sync_copy(map_ref.at[pl.ds(base, _DD_CH)], idx_d)
            pltpu.sync_copy(rows_ref.at[pl.ds(base, _DD_CH)], rbuf)
            for k in range(_DD_CH // 16):
                dv = idx_d[pl.ds(k * 16, 16)]
                m = (dv >= lo) & (dv < lo + _DD_HALF)
                idx_d[pl.ds(k * 16, 16)] = jnp.where(
                    m, dv - lo, jnp.full((16,), _DD_HALF, jnp.int32))
            pltpu.sync_copy(rbuf, acc.at[idx_d], add=True)
            pltpu.sync_copy(ones_v, cnt.at[idx_d], add=True)
            return 0

        n_i = (_DD_NCHUNK - s + NS - 1) // NS
        lax.fori_loop(0, n_i, one, 0)
        plsc.subcore_barrier()

        def flush(j, _):
            off = s * 320 + j * 40
            pltpu.sync_copy(acc.at[pl.ds(off, 40)], rbuf.at[pl.ds(0, 40)])
            pltpu.sync_copy(cnt.at[pl.ds(off, 40)], cbuf.at[pl.ds(0, 40)])

            def scale(r, _):
                inv = 1.0 / jnp.maximum(cbuf[r, pl.ds(0, 16)], 1.0)
                for v in range(8):
                    rbuf[r, pl.ds(v * 16, 16)] = rbuf[r, pl.ds(v * 16, 16)] * inv
                return 0

            lax.fori_loop(0, 40, scale, 0)
            pltpu.sync_copy(rbuf.at[pl.ds(0, 40)],
                            out_ref.at[pl.ds(lo + off, 40)])
            return 0

        n_f = jnp.where(s == 15, 5, 8)
        lax.fori_loop(0, n_f, flush, 0)

    fn = pl.kernel(
        body,
        mesh=_mesh(),
        out_type=jax.ShapeDtypeStruct((N, NHID), jnp.float32),
        scratch_types=[
            pltpu.VMEM((_DD_CH,), jnp.int32),
            pltpu.VMEM((_DD_CH, NHID), jnp.float32),
            pltpu.VMEM((_DD_CH, NHID), jnp.float32),
            pltpu.VMEM((_DD_CH, NHID), jnp.float32),
            pltpu.VMEM_SHARED((_DD_ACC, NHID), jnp.float32),
            pltpu.VMEM_SHARED((_DD_ACC, NHID), jnp.float32),
            pltpu.SemaphoreType.DMA,
        ],
    )
    return fn(rows, mapper)


def _tc_lin(x, W, b, blk):
    R, K = x.shape
    Kw, F = W.shape
    assert R % blk == 0

    def body(x_ref, w_ref, b_ref, o_ref):
        o_ref[...] = jax.lax.dot_general(
            x_ref[...], w_ref[...], (((1,), (0,)), ((), ())),
            preferred_element_type=jnp.float32) + b_ref[...]

    return pl.pallas_call(
        body,
        grid=(R // blk,),
        in_specs=[
            pl.BlockSpec((blk, K), lambda i: (i, 0)),
            pl.BlockSpec((Kw, F), lambda i: (0, 0)),
            pl.BlockSpec((1, F), lambda i: (0, 0)),
        ],
        out_specs=pl.BlockSpec((blk, F), lambda i: (i, 0)),
        out_shape=jax.ShapeDtypeStruct((R, F), jnp.float32),
    )(x, W, b.reshape(1, F))


_GNN_BLK = 400


def _tc_gnn_mlp(h, aggr, W1, b1, W2, b2, eps_i):
    def body(h_ref, a_ref, w1_ref, b1_ref, w2_ref, b2_ref, e_ref, o_ref):
        ep = e_ref[0, 0]
        z = (1.0 + ep) * h_ref[...] + a_ref[...]
        z = jax.lax.dot_general(z, w1_ref[...], (((1,), (0,)), ((), ())),
                                preferred_element_type=jnp.float32) + b1_ref[...]
        z = jnp.maximum(z, 0.0)
        z = jax.lax.dot_general(z, w2_ref[...], (((1,), (0,)), ((), ())),
                                preferred_element_type=jnp.float32) + b2_ref[...]
        o_ref[...] = h_ref[...] + jnp.maximum(z, 0.0)

    return pl.pallas_call(
        body,
        grid=(N_SUB // _GNN_BLK,),
        in_specs=[
            pl.BlockSpec((_GNN_BLK, NHID), lambda i: (i, 0)),
            pl.BlockSpec((_GNN_BLK, NHID), lambda i: (i, 0)),
            pl.BlockSpec((NHID, NHID), lambda i: (0, 0)),
            pl.BlockSpec((1, NHID), lambda i: (0, 0)),
            pl.BlockSpec((NHID, NHID), lambda i: (0, 0)),
            pl.BlockSpec((1, NHID), lambda i: (0, 0)),
            pl.BlockSpec((1, 1), lambda i: (0, 0)),
        ],
        out_specs=pl.BlockSpec((_GNN_BLK, NHID), lambda i: (i, 0)),
        out_shape=jax.ShapeDtypeStruct((N_SUB, NHID), jnp.float32),
    )(h, aggr, W1, b1.reshape(1, NHID), W2, b2.reshape(1, NHID),
      eps_i.reshape(1, 1))


def _tc_patch_pool(h, ids3, inv_c, W_u, b_u):
    nblk = N_SUB // _GNN_BLK

    def body(h_ref, id_ref, ic_ref, wu_ref, bu_ref, mean_ref, u_ref, s_acc):
        i = pl.program_id(0)

        @pl.when(i == 0)
        def _():
            s_acc[...] = jnp.zeros((BP, NHID), jnp.float32)

        ids = id_ref[0].reshape(_GNN_BLK, 1)
        onehot = (ids == jax.lax.broadcasted_iota(
            jnp.int32, (_GNN_BLK, BP), 1)).astype(jnp.float32)
        s_acc[...] += jax.lax.dot_general(
            onehot, h_ref[...], (((0,), (0,)), ((), ())),
            preferred_element_type=jnp.float32)

        @pl.when(i == nblk - 1)
        def _():
            mean = s_acc[...] * ic_ref[...]
            mean_ref[...] = mean
            u = jax.lax.dot_general(mean, wu_ref[...], (((1,), (0,)), ((), ())),
                                    preferred_element_type=jnp.float32)
            u_ref[...] = jnp.maximum(u + bu_ref[...], 0.0)

    return pl.pallas_call(
        body,
        grid=(nblk,),
        in_specs=[
            pl.BlockSpec((_GNN_BLK, NHID), lambda i: (i, 0)),
            pl.BlockSpec((1, 1, _GNN_BLK), lambda i: (i, 0, 0)),
            pl.BlockSpec((BP, 1), lambda i: (0, 0)),
            pl.BlockSpec((NHID, NHID), lambda i: (0, 0)),
            pl.BlockSpec((1, NHID), lambda i: (0, 0)),
        ],
        out_specs=[
            pl.BlockSpec((BP, NHID), lambda i: (0, 0)),
            pl.BlockSpec((BP, NHID), lambda i: (0, 0)),
        ],
        out_shape=[
            jax.ShapeDtypeStruct((BP, NHID), jnp.float32),
            jax.ShapeDtypeStruct((BP, NHID), jnp.float32),
        ],
        scratch_shapes=[pltpu.VMEM((BP, NHID), jnp.float32)],
    )(h, ids3, inv_c.reshape(BP, 1), W_u, b_u.reshape(1, NHID))


def _tc_broadcast_add(h, ids3, u):
    def body(h_ref, id_ref, u_ref, o_ref):
        ids = id_ref[0].reshape(_GNN_BLK, 1)
        onehot = (ids == jax.lax.broadcasted_iota(
            jnp.int32, (_GNN_BLK, BP), 1)).astype(jnp.float32)
        o_ref[...] = h_ref[...] + jax.lax.dot_general(
            onehot, u_ref[...], (((1,), (0,)), ((), ())),
            preferred_element_type=jnp.float32)

    return pl.pallas_call(
        body,
        grid=(N_SUB // _GNN_BLK,),
        in_specs=[
            pl.BlockSpec((_GNN_BLK, NHID), lambda i: (i, 0)),
            pl.BlockSpec((1, 1, _GNN_BLK), lambda i: (i, 0, 0)),
            pl.BlockSpec((BP, NHID), lambda i: (0, 0)),
        ],
        out_specs=pl.BlockSpec((_GNN_BLK, NHID), lambda i: (i, 0)),
        out_shape=jax.ShapeDtypeStruct((N_SUB, NHID), jnp.float32),
    )(h, ids3, u)


def _tc_mixer_head(sub_mean, mask_row,
                   W_t1, b_t1, W_t2, b_t2, W_c1, b_c1, W_c2, b_c2,
                   ln1_g, ln1_b, ln2_g, ln2_b, W_o1, b_o1, W_o2, b_o2):
    def ln(x, g, b):
        m = jnp.mean(x, axis=-1, keepdims=True)
        v = jnp.mean((x - m) ** 2, axis=-1, keepdims=True)
        return (x - m) / jnp.sqrt(v + 1e-5) * g + b

    def mm(a, bmat):
        return jax.lax.dot_general(a, bmat, (((1,), (0,)), ((), ())),
                                   preferred_element_type=jnp.float32)

    def body(x_ref, mk_ref, wt1_ref, bt1_ref, wt2_ref, bt2_ref,
             wc1_ref, bc1_ref, wc2_ref, bc2_ref,
             l1g_ref, l1b_ref, l2g_ref, l2b_ref,
             wo1_ref, bo1_ref, wo2_ref, bo2_ref, o_ref):
        xcur = x_ref[...]
        for i in range(NLAYER_MIX):
            y = ln(xcur, l1g_ref[i], l1b_ref[i])
            parts = []
            for bi in range(B):
                yb = y[bi * P:(bi + 1) * P, :]
                z = yb.T
                z = jnp.maximum(mm(z, wt1_ref[i]) + bt1_ref[i], 0.0)
                z = mm(z, wt2_ref[i]) + bt2_ref[i]
                parts.append(z.T)
            xcur = xcur + jnp.concatenate(parts, axis=0)
            y = ln(xcur, l2g_ref[i], l2b_ref[i])
            y = jnp.maximum(mm(y, wc1_ref[i]) + bc1_ref[i], 0.0)
            xcur = xcur + mm(y, wc2_ref[i]) + bc2_ref[i]
        mk = mk_ref[...].reshape(BP, 1)
        w = xcur * mk
        pooled = []
        for bi in range(B):
            seg = w[bi * P:(bi + 1) * P, :]
            den = jnp.sum(mk[bi * P:(bi + 1) * P, :])
            pooled.append(jnp.sum(seg, axis=0, keepdims=True) /
                          jnp.maximum(den, 1e-9))
        pooled = jnp.concatenate(pooled, axis=0)
        z = jnp.maximum(mm(pooled, wo1_ref[...]) + bo1_ref[...], 0.0)
        out = mm(z, wo2_ref[...]) + bo2_ref[...]
        o_ref[...] = jnp.pad(out, ((0, 4), (0, 64)))

    args = (sub_mean, mask_row, W_t1, b_t1, W_t2, b_t2, W_c1, b_c1, W_c2,
            b_c2, ln1_g, ln1_b, ln2_g, ln2_b, W_o1, b_o1.reshape(1, NHID),
            W_o2, b_o2.reshape(1, 64))
    return pl.pallas_call(
        body,
        out_shape=jax.ShapeDtypeStruct((8, 128), jnp.float32),
    )(*args)


def kernel(x, edge_attr, combined_subgraphs, subgraphs_nodes_mapper,
           subgraphs_edges_mapper, subgraphs_batch, mask, W_in, b_in,
           W_edge, b_edge, W_g, b_g, W_g2, b_g2, eps, W_u, b_u,
           W_t1, b_t1, W_t2, b_t2, W_c1, b_c1, W_c2, b_c2,
           ln1_g, ln1_b, ln2_g, ln2_b, W_o1, b_o1, W_o2, b_o2):
    src = combined_subgraphs[0].astype(jnp.int32)
    dst = combined_subgraphs[1].astype(jnp.int32)
    nmap = subgraphs_nodes_mapper.astype(jnp.int32)
    emap = subgraphs_edges_mapper.astype(jnp.int32)
    batch = subgraphs_batch.astype(jnp.int32)

    idx_all = jnp.stack([src.reshape(_CONV_NCHUNK, _CONV_CH),
                         dst.reshape(_CONV_NCHUNK, _CONV_CH),
                         emap.reshape(_CONV_NCHUNK, _CONV_CH)], axis=1)
    bounds = jnp.searchsorted(batch, jnp.arange(BP + 1, dtype=jnp.int32))
    c = (bounds[1:] - bounds[:-1]).astype(jnp.float32)
    inv_c = 1.0 / jnp.maximum(c, 1.0)
    ids3 = batch.reshape(N_SUB // _GNN_BLK, 1, _GNN_BLK)

    h0 = _tc_lin(x, W_in, b_in, blk=400)
    h = _sc_gather(h0, nmap, N_SUB, chunk=80)
    e_all = _tc_lin(edge_attr, W_edge, b_edge, blk=640)

    for i in range(NLAYER_GNN):
        if i > 0:
            _, u = _tc_patch_pool(h, ids3, inv_c, W_u, b_u)
            h = _tc_broadcast_add(h, ids3, u)
            means = _sc_dedup_mean(h, nmap)
            h = _sc_gather(means, nmap, N_SUB, chunk=80)
        aggr = _sc_conv(h, e_all, idx_all)
        h = _tc_gnn_mlp(h, aggr, W_g[i], b_g[i], W_g2[i], b_g2[i], eps[i])

    sub_mean, _ = _tc_patch_pool(h, ids3, inv_c, W_u, b_u)
    out_pad = _tc_mixer_head(
        sub_mean, mask.reshape(1, BP), W_t1, b_t1, W_t2, b_t2,
        W_c1, b_c1, W_c2, b_c2, ln1_g, ln1_b, ln2_g, ln2_b,
        W_o1, b_o1, W_o2, b_o2)
    return out_pad[:B, :64]


# speedup vs baseline: 2.8167x; 1.0244x over previous
import functools

import jax
import jax.numpy as jnp
from jax import lax
from jax.experimental import pallas as pl
from jax.experimental.pallas import tpu as pltpu
from jax.experimental.pallas import tpu_sc as plsc

N = 10000
E = 160000
N_SUB = 20000
E_SUB = 320000
B = 4
P = 32
BP = B * P
NHID = 128
NFEAT_EDGE = 16
NLAYER_GNN = 2
NLAYER_MIX = 2

@functools.lru_cache(maxsize=1)
def _mesh():
    return plsc.VectorSubcoreMesh(core_axis_name="c", subcore_axis_name="s")


NC = 2
NS = 16
NW = NC * NS


def _sc_gather(table, idx, rows, chunk):
    V, D = table.shape
    assert rows % chunk == 0
    nchunks = rows // chunk

    def body(tab_ref, idx_ref, out_ref, idx_v, rows_v, sem):
        c = lax.axis_index("c")
        s = lax.axis_index("s")
        w = s * NC + c

        def one(i, _):
            ci = w + i * NW
            base = ci * chunk
            pltpu.sync_copy(idx_ref.at[pl.ds(base, chunk)], idx_v)
            pltpu.async_copy(tab_ref.at[idx_v], rows_v, sem).wait()
            pltpu.sync_copy(rows_v, out_ref.at[pl.ds(base, chunk)])
            return 0

        n_i = (nchunks - w + NW - 1) // NW
        lax.fori_loop(0, n_i, one, 0)

    fn = pl.kernel(
        body,
        mesh=_mesh(),
        out_type=jax.ShapeDtypeStruct((rows, D), jnp.float32),
        scratch_types=[
            pltpu.VMEM((chunk,), jnp.int32),
            pltpu.VMEM((chunk, D), jnp.float32),
            pltpu.SemaphoreType.DMA,
        ],
    )
    return fn(table, idx)


_CONV_CH = 80
_CONV_NCHUNK = E_SUB // _CONV_CH
_CONV_HALF = N_SUB // NC
_CONV_ACC = 10240


def _sc_conv(h, e_all, idx_all):
    def body(h_ref, e_ref, ix_ref, out_ref,
             ix0, ix1, sx0, sx1, hr0, hr1, er0, er1, acc,
             si0, si1, sh0, sh1, se0, se1, sc0, sc1):
        c = lax.axis_index("c")
        s = lax.axis_index("s")
        lo = c * _CONV_HALF
        ixs = (ix0, ix1)
        sxs = (sx0, sx1)
        hrs = (hr0, hr1)
        ers = (er0, er1)
        sis = (si0, si1)
        shs = (sh0, sh1)
        ses = (se0, se1)
        scs = (sc0, sc1)

        def zrow(r, _):
            for v in range(8):
                hr0[r, pl.ds(v * 16, 16)] = jnp.zeros((16,), jnp.float32)
            return 0

        lax.fori_loop(0, _CONV_CH, zrow, 0)
        for j in range(640 // _CONV_CH):
            pltpu.sync_copy(hr0, acc.at[pl.ds(s * 640 + j * _CONV_CH, _CONV_CH)])
        plsc.subcore_barrier()

        n_i = (_CONV_NCHUNK - s + NS - 1) // NS

        def fire_idx(g, b):
            @pl.when(g < n_i)
            def _():
                pltpu.async_copy(ix_ref.at[s + g * NS], ixs[b], sis[b])

        def fire_gathers(g, b):
            @pl.when(g < n_i)
            def _():
                pltpu.make_async_copy(ix_ref.at[s + g * NS], ixs[b], sis[b]).wait()
                pltpu.async_copy(h_ref.at[ixs[b].at[0]], hrs[b], shs[b])
                pltpu.async_copy(e_ref.at[ixs[b].at[2]], ers[b], ses[b])

        def step(g, b):
            @pl.when(g < n_i)
            def _():
                ix, sx, hr, er = ixs[b], sxs[b], hrs[b], ers[b]
                pltpu.make_async_copy(h_ref.at[ix.at[0]], hr, shs[b]).wait()
                pltpu.make_async_copy(e_ref.at[ix.at[2]], er, ses[b]).wait()
                for k in range(_CONV_CH // 16):
                    dv = ix[1, pl.ds(k * 16, 16)]
                    m = (dv >= lo) & (dv < lo + _CONV_HALF)
                    sx[pl.ds(k * 16, 16)] = jnp.where(
                        m, dv - lo, jnp.full((16,), _CONV_HALF, jnp.int32))
                fire_idx(g + 2, b)
                @pl.when(g >= 1)
                def _():
                    pltpu.make_async_copy(
                        ers[1 - b], acc.at[sxs[1 - b]], scs[1 - b]).wait()
                fire_gathers(g + 1, 1 - b)
                def relu_row(r, _):
                    for v in range(8):
                        hv = hr[r, pl.ds(v * 16, 16)]
                        ev = er[r, pl.ds(v * 16, 16)]
                        er[r, pl.ds(v * 16, 16)] = jnp.maximum(hv + ev, 0.0)
                    return 0

                lax.fori_loop(0, _CONV_CH, relu_row, 0)
                pltpu.async_copy(er, acc.at[sx], scs[b], add=True)

        fire_idx(0, 0)
        fire_idx(1, 1)
        fire_gathers(0, 0)

        def pair(j, _):
            step(2 * j, 0)
            step(2 * j + 1, 1)
            return 0

        lax.fori_loop(0, (n_i + 1) // 2, pair, 0)

        @pl.when((n_i % 2) == 1)
        def _():
            pltpu.make_async_copy(er0, acc.at[sx0], sc0).wait()

        @pl.when((n_i % 2) == 0)
        def _():
            pltpu.make_async_copy(er1, acc.at[sx1], sc1).wait()

        plsc.subcore_barrier()

        @pl.when(s < 15)
        def _():
            pltpu.sync_copy(acc.at[pl.ds(s * 640, 640)],
                            out_ref.at[pl.ds(lo + s * 640, 640)])

        @pl.when(s == 15)
        def _():
            pltpu.sync_copy(acc.at[pl.ds(9600, 400)],
                            out_ref.at[pl.ds(lo + 9600, 400)])

    fn = pl.kernel(
        body,
        mesh=_mesh(),
        out_type=jax.ShapeDtypeStruct((N_SUB, NHID), jnp.float32),
        scratch_types=[
            pltpu.VMEM((3, _CONV_CH), jnp.int32),
            pltpu.VMEM((3, _CONV_CH), jnp.int32),
            pltpu.VMEM((_CONV_CH,), jnp.int32),
            pltpu.VMEM((_CONV_CH,), jnp.int32),
            pltpu.VMEM((_CONV_CH, NHID), jnp.float32),
            pltpu.VMEM((_CONV_CH, NHID), jnp.float32),
            pltpu.VMEM((_CONV_CH, NHID), jnp.float32),
            pltpu.VMEM((_CONV_CH, NHID), jnp.float32),
            pltpu.VMEM_SHARED((_CONV_ACC, NHID), jnp.float32),
            pltpu.SemaphoreType.DMA,
            pltpu.SemaphoreType.DMA,
            pltpu.SemaphoreType.DMA,
            pltpu.SemaphoreType.DMA,
            pltpu.SemaphoreType.DMA,
            pltpu.SemaphoreType.DMA,
            pltpu.SemaphoreType.DMA,
            pltpu.SemaphoreType.DMA,
        ],
    )
    return fn(h, e_all, idx_all)


_DD_CH = 80
_DD_NCHUNK = N_SUB // _DD_CH
_DD_HALF = N // NC
_DD_ACC = 5120


def _sc_dedup_mean(rows, mapper):
    def body(rows_ref, map_ref, out_ref,
             idx_d, rbuf, ones_v, cbuf, acc, cnt, sem):
        c = lax.axis_index("c")
        s = lax.axis_index("s")
        lo = c * _DD_HALF

        def zrow(r, _):
            for v in range(8):
                rbuf[r, pl.ds(v * 16, 16)] = jnp.zeros((16,), jnp.float32)
                ones_v[r, pl.ds(v * 16, 16)] = jnp.ones((16,), jnp.float32)
            return 0

        lax.fori_loop(0, _DD_CH, zrow, 0)
        for j in range(4):
            pltpu.sync_copy(rbuf, acc.at[pl.ds(s * 320 + j * 80, 80)])
            pltpu.sync_copy(rbuf, cnt.at[pl.ds(s * 320 + j * 80, 80)])
        plsc.subcore_barrier()

        def one(i, _):
            ci = s + i * NS
            base = ci * _DD_CH
            pltpu.sync_copy(map_ref.at[pl.ds(base, _DD_CH)], idx_d)
            pltpu.sync_copy(rows_ref.at[pl.ds(base, _DD_CH)], rbuf)
            for k in range(_DD_CH // 16):
                dv = idx_d[pl.ds(k * 16, 16)]
                m = (dv >= lo) & (dv < lo + _DD_HALF)
                idx_d[pl.ds(k * 16, 16)] = jnp.where(
                    m, dv - lo, jnp.full((16,), _DD_HALF, jnp.int32))
            pltpu.sync_copy(rbuf, acc.at[idx_d], add=True)
            pltpu.sync_copy(ones_v, cnt.at[idx_d], add=True)
            return 0

        n_i = (_DD_NCHUNK - s + NS - 1) // NS
        lax.fori_loop(0, n_i, one, 0)
        plsc.subcore_barrier()

        def flush(j, _):
            off = s * 320 + j * 40
            pltpu.sync_copy(acc.at[pl.ds(off, 40)], rbuf.at[pl.ds(0, 40)])
            pltpu.sync_copy(cnt.at[pl.ds(off, 40)], cbuf.at[pl.ds(0, 40)])

            def scale(r, _):
                inv = 1.0 / jnp.maximum(cbuf[r, pl.ds(0, 16)], 1.0)
                for v in range(8):
                    rbuf[r, pl.ds(v * 16, 16)] = rbuf[r, pl.ds(v * 16, 16)] * inv
                return 0

            lax.fori_loop(0, 40, scale, 0)
            pltpu.sync_copy(rbuf.at[pl.ds(0, 40)],
                            out_ref.at[pl.ds(lo + off, 40)])
            return 0

        n_f = jnp.where(s == 15, 5, 8)
        lax.fori_loop(0, n_f, flush, 0)

    fn = pl.kernel(
        body,
        mesh=_mesh(),
        out_type=jax.ShapeDtypeStruct((N, NHID), jnp.float32),
        scratch_types=[
            pltpu.VMEM((_DD_CH,), jnp.int32),
            pltpu.VMEM((_DD_CH, NHID), jnp.float32),
            pltpu.VMEM((_DD_CH, NHID), jnp.float32),
            pltpu.VMEM((_DD_CH, NHID), jnp.float32),
            pltpu.VMEM_SHARED((_DD_ACC, NHID), jnp.float32),
            pltpu.VMEM_SHARED((_DD_ACC, NHID), jnp.float32),
            pltpu.SemaphoreType.DMA,
        ],
    )
    return fn(rows, mapper)


def _tc_lin(x, W, b, blk):
    R, K = x.shape
    Kw, F = W.shape
    assert R % blk == 0

    def body(x_ref, w_ref, b_ref, o_ref):
        o_ref[...] = jax.lax.dot_general(
            x_ref[...], w_ref[...], (((1,), (0,)), ((), ())),
            preferred_element_type=jnp.float32) + b_ref[...]

    return pl.pallas_call(
        body,
        grid=(R // blk,),
        in_specs=[
            pl.BlockSpec((blk, K), lambda i: (i, 0)),
            pl.BlockSpec((Kw, F), lambda i: (0, 0)),
            pl.BlockSpec((1, F), lambda i: (0, 0)),
        ],
        out_specs=pl.BlockSpec((blk, F), lambda i: (i, 0)),
        out_shape=jax.ShapeDtypeStruct((R, F), jnp.float32),
    )(x, W, b.reshape(1, F))


_GNN_BLK = 400


def _tc_gnn_mlp(h, aggr, W1, b1, W2, b2, eps_i):
    def body(h_ref, a_ref, w1_ref, b1_ref, w2_ref, b2_ref, e_ref, o_ref):
        ep = e_ref[0, 0]
        z = (1.0 + ep) * h_ref[...] + a_ref[...]
        z = jax.lax.dot_general(z, w1_ref[...], (((1,), (0,)), ((), ())),
                                preferred_element_type=jnp.float32) + b1_ref[...]
        z = jnp.maximum(z, 0.0)
        z = jax.lax.dot_general(z, w2_ref[...], (((1,), (0,)), ((), ())),
                                preferred_element_type=jnp.float32) + b2_ref[...]
        o_ref[...] = h_ref[...] + jnp.maximum(z, 0.0)

    return pl.pallas_call(
        body,
        grid=(N_SUB // _GNN_BLK,),
        in_specs=[
            pl.BlockSpec((_GNN_BLK, NHID), lambda i: (i, 0)),
            pl.BlockSpec((_GNN_BLK, NHID), lambda i: (i, 0)),
            pl.BlockSpec((NHID, NHID), lambda i: (0, 0)),
            pl.BlockSpec((1, NHID), lambda i: (0, 0)),
            pl.BlockSpec((NHID, NHID), lambda i: (0, 0)),
            pl.BlockSpec((1, NHID), lambda i: (0, 0)),
            pl.BlockSpec((1, 1), lambda i: (0, 0)),
        ],
        out_specs=pl.BlockSpec((_GNN_BLK, NHID), lambda i: (i, 0)),
        out_shape=jax.ShapeDtypeStruct((N_SUB, NHID), jnp.float32),
    )(h, aggr, W1, b1.reshape(1, NHID), W2, b2.reshape(1, NHID),
      eps_i.reshape(1, 1))


def _tc_patch_pool(h, ids3, inv_c, W_u, b_u):
    nblk = N_SUB // _GNN_BLK

    def body(h_ref, id_ref, ic_ref, wu_ref, bu_ref, mean_ref, u_ref, s_acc):
        i = pl.program_id(0)

        @pl.when(i == 0)
        def _():
            s_acc[...] = jnp.zeros((BP, NHID), jnp.float32)

        ids = id_ref[0].reshape(_GNN_BLK, 1)
        onehot = (ids == jax.lax.broadcasted_iota(
            jnp.int32, (_GNN_BLK, BP), 1)).astype(jnp.float32)
        s_acc[...] += jax.lax.dot_general(
            onehot, h_ref[...], (((0,), (0,)), ((), ())),
            preferred_element_type=jnp.float32)

        @pl.when(i == nblk - 1)
        def _():
            mean = s_acc[...] * ic_ref[...]
            mean_ref[...] = mean
            u = jax.lax.dot_general(mean, wu_ref[...], (((1,), (0,)), ((), ())),
                                    preferred_element_type=jnp.float32)
            u_ref[...] = jnp.maximum(u + bu_ref[...], 0.0)

    return pl.pallas_call(
        body,
        grid=(nblk,),
        in_specs=[
            pl.BlockSpec((_GNN_BLK, NHID), lambda i: (i, 0)),
            pl.BlockSpec((1, 1, _GNN_BLK), lambda i: (i, 0, 0)),
            pl.BlockSpec((BP, 1), lambda i: (0, 0)),
            pl.BlockSpec((NHID, NHID), lambda i: (0, 0)),
            pl.BlockSpec((1, NHID), lambda i: (0, 0)),
        ],
        out_specs=[
            pl.BlockSpec((BP, NHID), lambda i: (0, 0)),
            pl.BlockSpec((BP, NHID), lambda i: (0, 0)),
        ],
        out_shape=[
            jax.ShapeDtypeStruct((BP, NHID), jnp.float32),
            jax.ShapeDtypeStruct((BP, NHID), jnp.float32),
        ],
        scratch_shapes=[pltpu.VMEM((BP, NHID), jnp.float32)],
    )(h, ids3, inv_c.reshape(BP, 1), W_u, b_u.reshape(1, NHID))


def _tc_broadcast_add(h, ids3, u):
    def body(h_ref, id_ref, u_ref, o_ref):
        ids = id_ref[0].reshape(_GNN_BLK, 1)
        onehot = (ids == jax.lax.broadcasted_iota(
            jnp.int32, (_GNN_BLK, BP), 1)).astype(jnp.float32)
        o_ref[...] = h_ref[...] + jax.lax.dot_general(
            onehot, u_ref[...], (((1,), (0,)), ((), ())),
            preferred_element_type=jnp.float32)

    return pl.pallas_call(
        body,
        grid=(N_SUB // _GNN_BLK,),
        in_specs=[
            pl.BlockSpec((_GNN_BLK, NHID), lambda i: (i, 0)),
            pl.BlockSpec((1, 1, _GNN_BLK), lambda i: (i, 0, 0)),
            pl.BlockSpec((BP, NHID), lambda i: (0, 0)),
        ],
        out_specs=pl.BlockSpec((_GNN_BLK, NHID), lambda i: (i, 0)),
        out_shape=jax.ShapeDtypeStruct((N_SUB, NHID), jnp.float32),
    )(h, ids3, u)


def _tc_mixer_head(sub_mean, mask_row,
                   W_t1, b_t1, W_t2, b_t2, W_c1, b_c1, W_c2, b_c2,
                   ln1_g, ln1_b, ln2_g, ln2_b, W_o1, b_o1, W_o2, b_o2):
    def ln(x, g, b):
        m = jnp.mean(x, axis=-1, keepdims=True)
        v = jnp.mean((x - m) ** 2, axis=-1, keepdims=True)
        return (x - m) / jnp.sqrt(v + 1e-5) * g + b

    def mm(a, bmat):
        return jax.lax.dot_general(a, bmat, (((1,), (0,)), ((), ())),
                                   preferred_element_type=jnp.float32)

    def body(x_ref, mk_ref, wt1_ref, bt1_ref, wt2_ref, bt2_ref,
             wc1_ref, bc1_ref, wc2_ref, bc2_ref,
             l1g_ref, l1b_ref, l2g_ref, l2b_ref,
             wo1_ref, bo1_ref, wo2_ref, bo2_ref, o_ref):
        xcur = x_ref[...]
        for i in range(NLAYER_MIX):
            y = ln(xcur, l1g_ref[i], l1b_ref[i])
            parts = []
            for bi in range(B):
                yb = y[bi * P:(bi + 1) * P, :]
                z = yb.T
                z = jnp.maximum(mm(z, wt1_ref[i]) + bt1_ref[i], 0.0)
                z = mm(z, wt2_ref[i]) + bt2_ref[i]
                parts.append(z.T)
            xcur = xcur + jnp.concatenate(parts, axis=0)
            y = ln(xcur, l2g_ref[i], l2b_ref[i])
            y = jnp.maximum(mm(y, wc1_ref[i]) + bc1_ref[i], 0.0)
            xcur = xcur + mm(y, wc2_ref[i]) + bc2_ref[i]
        mk = mk_ref[...].reshape(BP, 1)
        w = xcur * mk
        pooled = []
        for bi in range(B):
            seg = w[bi * P:(bi + 1) * P, :]
            den = jnp.sum(mk[bi * P:(bi + 1) * P, :])
            pooled.append(jnp.sum(seg, axis=0, keepdims=True) /
                          jnp.maximum(den, 1e-9))
        pooled = jnp.concatenate(pooled, axis=0)
        z = jnp.maximum(mm(pooled, wo1_ref[...]) + bo1_ref[...], 0.0)
        out = mm(z, wo2_ref[...]) + bo2_ref[...]
        o_ref[...] = jnp.pad(out, ((0, 4), (0, 64)))

    args = (sub_mean, mask_row, W_t1, b_t1, W_t2, b_t2, W_c1, b_c1, W_c2,
            b_c2, ln1_g, ln1_b, ln2_g, ln2_b, W_o1, b_o1.reshape(1, NHID),
            W_o2, b_o2.reshape(1, 64))
    return pl.pallas_call(
        body,
        out_shape=jax.ShapeDtypeStruct((8, 128), jnp.float32),
    )(*args)


def kernel(x, edge_attr, combined_subgraphs, subgraphs_nodes_mapper,
           subgraphs_edges_mapper, subgraphs_batch, mask, W_in, b_in,
           W_edge, b_edge, W_g, b_g, W_g2, b_g2, eps, W_u, b_u,
           W_t1, b_t1, W_t2, b_t2, W_c1, b_c1, W_c2, b_c2,
           ln1_g, ln1_b, ln2_g, ln2_b, W_o1, b_o1, W_o2, b_o2):
    src = combined_subgraphs[0].astype(jnp.int32)
    dst = combined_subgraphs[1].astype(jnp.int32)
    nmap = subgraphs_nodes_mapper.astype(jnp.int32)
    emap = subgraphs_edges_mapper.astype(jnp.int32)
    batch = subgraphs_batch.astype(jnp.int32)

    idx_all = jnp.stack([src.reshape(_CONV_NCHUNK, _CONV_CH),
                         dst.reshape(_CONV_NCHUNK, _CONV_CH),
                         emap.reshape(_CONV_NCHUNK, _CONV_CH)], axis=1)
    bounds = jnp.searchsorted(batch, jnp.arange(BP + 1, dtype=jnp.int32))
    c = (bounds[1:] - bounds[:-1]).astype(jnp.float32)
    inv_c = 1.0 / jnp.maximum(c, 1.0)
    ids3 = batch.reshape(N_SUB // _GNN_BLK, 1, _GNN_BLK)

    h0 = _tc_lin(x, W_in, b_in, blk=400)
    h = _sc_gather(h0, nmap, N_SUB, chunk=80)
    e_all = _tc_lin(edge_attr, W_edge, b_edge, blk=640)

    for i in range(NLAYER_GNN):
        if i > 0:
            _, u = _tc_patch_pool(h, ids3, inv_c, W_u, b_u)
            h = _tc_broadcast_add(h, ids3, u)
            means = _sc_dedup_mean(h, nmap)
            h = _sc_gather(means, nmap, N_SUB, chunk=80)
        aggr = _sc_conv(h, e_all, idx_all)
        h = _tc_gnn_mlp(h, aggr, W_g[i], b_g[i], W_g2[i], b_g2[i], eps[i])

    sub_mean, _ = _tc_patch_pool(h, ids3, inv_c, W_u, b_u)
    out_pad = _tc_mixer_head(
        sub_mean, mask.reshape(1, BP), W_t1, b_t1, W_t2, b_t2,
        W_c1, b_c1, W_c2, b_c2, ln1_g, ln1_b, ln2_g, ln2_b,
        W_o1, b_o1, W_o2, b_o2)
    return out_pad[:B, :64]
